# Initial kernel scaffold; baseline (speedup 1.0000x reference)
import jax
import jax.numpy as jnp
from jax.experimental import pallas as pl


def kernel(x, W):
    raise NotImplementedError("write your pallas kernel here")


# speedup vs baseline: 2.1024x; 2.1024x over previous
import functools

import jax
import jax.numpy as jnp
from jax import lax
from jax.experimental import pallas as pl
from jax.experimental.pallas import tpu as pltpu
from jax.experimental.pallas import tpu_sc as plsc

_VOCAB = 100000
_EMBED = 128
_BATCH = 4096
_SEQ = 200

_NC = 2
_NS = 16
_NW = _NC * _NS
_IDX_TOTAL = _BATCH * _SEQ
_PER_W = _IDX_TOTAL // _NW
_ROWS = 128
_CHUNKS = _PER_W // _ROWS


def _sc_gather(x2d, W):
    mesh = plsc.VectorSubcoreMesh(core_axis_name="c", subcore_axis_name="s")

    @functools.partial(
        pl.kernel,
        mesh=mesh,
        out_type=jax.ShapeDtypeStruct((_IDX_TOTAL, _EMBED), jnp.float32),
        scratch_types=[
            pltpu.VMEM((_CHUNKS, _ROWS), jnp.int32),
            pltpu.VMEM((2, _ROWS, _EMBED), jnp.float32),
            pltpu.SemaphoreType.DMA,
        ],
    )
    def k(x_hbm, w_hbm, out_hbm, idx_v, rows_v, sem):
        wid = lax.axis_index("s") * _NC + lax.axis_index("c")
        pltpu.sync_copy(x_hbm.at[pl.ds(wid * _CHUNKS, _CHUNKS)], idx_v)
        base = wid * _PER_W

        def body(j, carry):
            pltpu.async_copy(w_hbm.at[idx_v.at[j]], rows_v.at[0], sem).wait()
            pltpu.sync_copy(
                rows_v.at[0], out_hbm.at[pl.ds(base + j * _ROWS, _ROWS)]
            )
            return carry

        lax.fori_loop(0, _CHUNKS, body, 0)

    return k(x2d, W)


def _tc_transpose(G):
    BB = 8

    def body(g_ref, o_ref):
        o_ref[...] = jnp.swapaxes(g_ref[...], 1, 2)

    return pl.pallas_call(
        body,
        grid=(_BATCH // BB,),
        in_specs=[pl.BlockSpec((BB, _SEQ, _EMBED), lambda i: (i, 0, 0))],
        out_specs=pl.BlockSpec((BB, _EMBED, _SEQ), lambda i: (i, 0, 0)),
        out_shape=jax.ShapeDtypeStruct((_BATCH, _EMBED, _SEQ), jnp.float32),
    )(G)


def kernel(x, W):
    x2d = x.reshape(_IDX_TOTAL // _ROWS, _ROWS)
    G = _sc_gather(x2d, W)
    return _tc_transpose(G.reshape(_BATCH, _SEQ, _EMBED))


# speedup vs baseline: 2.3450x; 1.1154x over previous
import functools

import jax
import jax.numpy as jnp
from jax import lax
from jax.experimental import pallas as pl
from jax.experimental.pallas import tpu as pltpu
from jax.experimental.pallas import tpu_sc as plsc

_VOCAB = 100000
_EMBED = 128
_BATCH = 4096
_SEQ = 200

_NC = 2
_NS = 16
_NW = _NC * _NS
_IDX_TOTAL = _BATCH * _SEQ
_PER_W = _IDX_TOTAL // _NW
_ROWS = 128
_CHUNKS = _PER_W // _ROWS


def _sc_gather(x2d, W):
    mesh = plsc.VectorSubcoreMesh(core_axis_name="c", subcore_axis_name="s")

    @functools.partial(
        pl.kernel,
        mesh=mesh,
        out_type=jax.ShapeDtypeStruct((_IDX_TOTAL, _EMBED), jnp.float32),
        scratch_types=[
            pltpu.VMEM((_CHUNKS, _ROWS), jnp.int32),
            pltpu.VMEM((2, _ROWS, _EMBED), jnp.float32),
            pltpu.SemaphoreType.DMA,
        ],
    )
    def k(x_hbm, w_hbm, out_hbm, idx_v, rows_v, sem):
        wid = lax.axis_index("s") * _NC + lax.axis_index("c")
        pltpu.sync_copy(x_hbm.at[pl.ds(wid * _CHUNKS, _CHUNKS)], idx_v)
        base = wid * _PER_W

        def body(j, carry):
            pltpu.async_copy(w_hbm.at[idx_v.at[j]], rows_v.at[0], sem).wait()
            pltpu.sync_copy(
                rows_v.at[0], out_hbm.at[pl.ds(base + j * _ROWS, _ROWS)]
            )
            return carry

        lax.fori_loop(0, _CHUNKS, body, 0)

    return k(x2d, W)


def _tc_transpose(G):
    BB = 16

    def body(g_ref, o_ref):
        o_ref[...] = jnp.swapaxes(g_ref[...], 1, 2)

    return pl.pallas_call(
        body,
        grid=(_BATCH // BB,),
        in_specs=[pl.BlockSpec((BB, _SEQ, _EMBED), lambda i: (i, 0, 0))],
        out_specs=pl.BlockSpec((BB, _EMBED, _SEQ), lambda i: (i, 0, 0)),
        out_shape=jax.ShapeDtypeStruct((_BATCH, _EMBED, _SEQ), jnp.float32),
    )(G)


def kernel(x, W):
    x2d = x.reshape(_IDX_TOTAL // _ROWS, _ROWS)
    G = _sc_gather(x2d, W)
    return _tc_transpose(G.reshape(_BATCH, _SEQ, _EMBED))
